# Initial kernel scaffold; baseline (speedup 1.0000x reference)
import functools

import jax
import jax.numpy as jnp
from jax import lax
from jax.experimental import pallas as pl
from jax.experimental.pallas import tpu as pltpu
from jax.experimental.pallas import tpu_sc as plsc

B, C, N, K = 4, 128, 4096, 16
S = N // 4
NR, NL = 32, 128
SR, SL = 8, 128

SC_CORES, SC_SUBCORES = 2, 16
NTILES = SC_CORES * SC_SUBCORES
PPT = (B * S) // NTILES
CPTS = 8
NCHUNK = PPT // CPTS


def _fps_body(x_ref, cent_ref):
    pid = (lax.broadcasted_iota(jnp.int32, (NR, NL), 0) * NL
           + lax.broadcasted_iota(jnp.int32, (NR, NL), 1))
    sid = (lax.broadcasted_iota(jnp.int32, (SR, SL), 0) * SL
           + lax.broadcasted_iota(jnp.int32, (SR, SL), 1))
    xs = [[x_ref[b, d] for d in range(3)] for b in range(B)]

    def body(i, carry):
        dists, cents, fids = carry
        new_dists, new_cents, new_fids = [], [], []
        for b in range(B):
            dist, cent, fid = dists[b], cents[b], fids[b]
            cent = jnp.where(sid == i, fid, cent)
            oh = pid == fid
            c0 = jnp.sum(jnp.where(oh, xs[b][0], 0.0), keepdims=True)
            c1 = jnp.sum(jnp.where(oh, xs[b][1], 0.0), keepdims=True)
            c2 = jnp.sum(jnp.where(oh, xs[b][2], 0.0), keepdims=True)
            d0 = xs[b][0] - c0
            d1 = xs[b][1] - c1
            d2 = xs[b][2] - c2
            dsq = d0 * d0 + d1 * d1 + d2 * d2
            dist = jnp.minimum(dist, dsq)
            mx = jnp.max(dist, keepdims=True)
            cand = jnp.where(dist == mx, pid, jnp.int32(N))
            nf = jnp.min(cand, keepdims=True)
            new_dists.append(dist)
            new_cents.append(cent)
            new_fids.append(nf)
        return tuple(new_dists), tuple(new_cents), tuple(new_fids)

    dists0 = tuple(jnp.full((NR, NL), 1e10, jnp.float32) for _ in range(B))
    cents0 = tuple(jnp.zeros((SR, SL), jnp.int32) for _ in range(B))
    fids0 = tuple(jnp.zeros((1, 1), jnp.int32) for _ in range(B))
    _, cents, _ = lax.fori_loop(0, S, body, (dists0, cents0, fids0))
    for b in range(B):
        cent_ref[b] = cents[b]


_sc_mesh = plsc.VectorSubcoreMesh(core_axis_name="c", subcore_axis_name="s")


@functools.partial(
    pl.kernel,
    mesh=_sc_mesh,
    out_type=(
        jax.ShapeDtypeStruct((B * S, C), jnp.float32),
        jax.ShapeDtypeStruct((B * S, 16), jnp.float32),
    ),
    scratch_types=[
        pltpu.VMEM((PPT,), jnp.int32),
        pltpu.VMEM((PPT, K), jnp.int32),
        pltpu.VMEM((CPTS * K,), jnp.int32),
        pltpu.VMEM((CPTS * K, C), jnp.float32),
        pltpu.VMEM((PPT, C), jnp.float32),
        pltpu.VMEM((PPT, 16), jnp.float32),
        pltpu.SemaphoreType.DMA,
    ],
)
def _sc_pool(gpt_hbm, idx2_hbm, feat_hbm, vtx_hbm, outf_hbm, outv_hbm,
             cent_v, nbr2d, fidx, rows, outf, vtxv, sem):
    wid = lax.axis_index("s") * SC_CORES + lax.axis_index("c")
    base = wid * PPT
    pltpu.sync_copy(gpt_hbm.at[pl.ds(base, PPT)], cent_v)
    pltpu.async_copy(idx2_hbm.at[cent_v], nbr2d, sem).wait()
    pltpu.async_copy(vtx_hbm.at[cent_v], vtxv, sem).wait()
    pltpu.sync_copy(vtxv, outv_hbm.at[pl.ds(base, PPT)])

    def chunk_body(ci, _):
        p0 = ci * CPTS
        for j in range(CPTS):
            fidx[pl.ds(j * K, K)] = nbr2d[p0 + j, :]
        pltpu.async_copy(feat_hbm.at[fidx], rows, sem).wait()
        for j in range(CPTS):
            for cg in range(C // 16):
                cs = pl.ds(cg * 16, 16)
                acc = rows[j * K, cs]
                for r in range(1, K):
                    acc = jnp.maximum(acc, rows[j * K + r, cs])
                outf[p0 + j, cs] = acc
        return 0

    lax.fori_loop(0, NCHUNK, chunk_body, 0)
    pltpu.sync_copy(outf, outf_hbm.at[pl.ds(base, PPT)])


def kernel(vertices, feature_map, idx):
    idx2 = idx.astype(jnp.int32).reshape(B * N, K)
    x4 = vertices.reshape(B, 3, NR, NL)
    cent = pl.pallas_call(
        _fps_body,
        out_shape=jax.ShapeDtypeStruct((B, SR, SL), jnp.int32),
    )(x4)
    cent2 = cent.reshape(B, S)
    gpt = (cent2 + (jnp.arange(B, dtype=jnp.int32) * N)[:, None]).reshape(B * S)
    flat_f = feature_map.transpose(0, 2, 1).reshape(B * N, C)
    vtx = jnp.concatenate(
        [vertices.transpose(0, 2, 1).reshape(B * N, 3),
         jnp.zeros((B * N, 13), jnp.float32)], axis=1)
    outf, outv = _sc_pool(gpt, idx2, flat_f, vtx)
    feature_map_pool = outf.reshape(B, S, C).transpose(0, 2, 1)
    vertices_pool = outv[:, :3].reshape(B, S, 3).transpose(0, 2, 1)
    return (vertices_pool, feature_map_pool)


# speedup vs baseline: 6.5331x; 6.5331x over previous
import functools

import jax
import jax.numpy as jnp
from jax import lax
from jax.experimental import pallas as pl
from jax.experimental.pallas import tpu as pltpu
from jax.experimental.pallas import tpu_sc as plsc

B, C, N, K = 4, 128, 4096, 16
S = N // 4
NR, NL = 32, 128
SR, SL = 8, 128

SC_CORES, SC_SUBCORES = 2, 16
NTILES = SC_CORES * SC_SUBCORES
PPT = (B * S) // NTILES
CPTS = 8
NCHUNK = PPT // CPTS
IDXCOLS = 128
PTS_PER_IDXROW = IDXCOLS // K


def _fps_body(x_ref, cent_ref, vp_ref):
    pid = (lax.broadcasted_iota(jnp.int32, (NR, NL), 0) * NL
           + lax.broadcasted_iota(jnp.int32, (NR, NL), 1))
    sid = (lax.broadcasted_iota(jnp.int32, (SR, SL), 0) * SL
           + lax.broadcasted_iota(jnp.int32, (SR, SL), 1))
    xs = [[x_ref[b, d] for d in range(3)] for b in range(B)]

    def body(i, carry):
        dists, cents, fids, vps = carry
        new_dists, new_cents, new_fids, new_vps = [], [], [], []
        for b in range(B):
            dist, cent, fid = dists[b], cents[b], fids[b]
            sel = sid == i
            cent = jnp.where(sel, fid, cent)
            oh = pid == fid
            cds = [jnp.sum(jnp.where(oh, xs[b][d], 0.0), keepdims=True)
                   for d in range(3)]
            vp = tuple(jnp.where(sel, cds[d], vps[b][d]) for d in range(3))
            d0 = xs[b][0] - cds[0]
            d1 = xs[b][1] - cds[1]
            d2 = xs[b][2] - cds[2]
            dsq = d0 * d0 + d1 * d1 + d2 * d2
            dist = jnp.minimum(dist, dsq)
            mx = jnp.max(dist, keepdims=True)
            cand = jnp.where(dist == mx, pid, jnp.int32(N))
            nf = jnp.min(cand, keepdims=True)
            new_dists.append(dist)
            new_cents.append(cent)
            new_fids.append(nf)
            new_vps.append(vp)
        return (tuple(new_dists), tuple(new_cents), tuple(new_fids),
                tuple(new_vps))

    dists0 = tuple(jnp.full((NR, NL), 1e10, jnp.float32) for _ in range(B))
    cents0 = tuple(jnp.zeros((SR, SL), jnp.int32) for _ in range(B))
    fids0 = tuple(jnp.zeros((1, 1), jnp.int32) for _ in range(B))
    vps0 = tuple(tuple(jnp.zeros((SR, SL), jnp.float32) for _ in range(3))
                 for _ in range(B))
    _, cents, _, vps = lax.fori_loop(0, S, body, (dists0, cents0, fids0, vps0))
    for b in range(B):
        cent_ref[b] = cents[b]
        for d in range(3):
            vp_ref[b, d] = vps[b][d]


def _sc_pool_body(gpt_hbm, idxv_hbm, feat_hbm, outf_hbm,
                  cent_v, qidx, offv, idxrows, fidx2d, rows, outf, sem):
    wid = lax.axis_index("s") * SC_CORES + lax.axis_index("c")
    base = wid * PPT
    pltpu.sync_copy(gpt_hbm.at[pl.ds(base, PPT)], cent_v)

    for g in range(PPT // 16):
        v = cent_v[pl.ds(g * 16, 16)]
        qidx[pl.ds(g * 16, 16)] = lax.shift_right_logical(
            v, jnp.full((16,), 3, jnp.int32))
        offv[pl.ds(g * 16, 16)] = (v & 7) * K
    pltpu.async_copy(idxv_hbm.at[qidx], idxrows, sem).wait()

    lane = lax.iota(jnp.int32, 16)
    for g in range(PPT // 16):
        rowvec = jnp.full((16,), g * 16, jnp.int32) + lane
        crow = jnp.full((16,), 2 * g, jnp.int32) + lax.shift_right_logical(
            lane, jnp.full((16,), 3, jnp.int32))
        ccol0 = lane & 7
        off = offv[pl.ds(g * 16, 16)]
        for r in range(K):
            v = plsc.load_gather(idxrows, [rowvec, off + r])
            plsc.store_scatter(fidx2d, [crow, ccol0 + r * 8], v)

    def chunk_body(ci, _):
        pltpu.async_copy(feat_hbm.at[fidx2d.at[ci]], rows, sem).wait()

        def point_body(p, _):
            for cg in range(C // 16):
                cs = pl.ds(cg * 16, 16)
                acc = rows[p, cs]
                for r in range(1, K):
                    acc = jnp.maximum(acc, rows[r * CPTS + p, cs])
                outf[ci * CPTS + p, cs] = acc
            return 0

        lax.fori_loop(0, CPTS, point_body, 0)
        return 0

    lax.fori_loop(0, NCHUNK, chunk_body, 0)
    pltpu.sync_copy(outf, outf_hbm.at[pl.ds(base, PPT)])


@functools.cache
def _get_sc_pool():
    mesh = plsc.VectorSubcoreMesh(core_axis_name="c", subcore_axis_name="s")
    return pl.kernel(
        _sc_pool_body,
        mesh=mesh,
        compiler_params=pltpu.CompilerParams(needs_layout_passes=False),
        out_type=jax.ShapeDtypeStruct((B * S, C), jnp.float32),
        scratch_types=[
            pltpu.VMEM((PPT,), jnp.int32),
            pltpu.VMEM((PPT,), jnp.int32),
            pltpu.VMEM((PPT,), jnp.int32),
            pltpu.VMEM((PPT, IDXCOLS), jnp.int32),
            pltpu.VMEM((NCHUNK, CPTS * K), jnp.int32),
            pltpu.VMEM((CPTS * K, C), jnp.float32),
            pltpu.VMEM((PPT, C), jnp.float32),
            pltpu.SemaphoreType.DMA,
        ],
    )


def kernel(vertices, feature_map, idx):
    idxv = idx.astype(jnp.int32).reshape((B * N * K) // IDXCOLS, IDXCOLS)
    x4 = vertices.reshape(B, 3, NR, NL)
    cent, vpool = pl.pallas_call(
        _fps_body,
        out_shape=(jax.ShapeDtypeStruct((B, SR, SL), jnp.int32),
                   jax.ShapeDtypeStruct((B, 3, SR, SL), jnp.float32)),
    )(x4)
    cent2 = cent.reshape(B, S)
    gpt = (cent2 + (jnp.arange(B, dtype=jnp.int32) * N)[:, None]).reshape(B * S)
    flat_f = feature_map.transpose(0, 2, 1).reshape(B * N, C)
    outf = _get_sc_pool()(gpt, idxv, flat_f)
    feature_map_pool = outf.reshape(B, S, C).transpose(0, 2, 1)
    vertices_pool = vpool.reshape(B, 3, S)
    return (vertices_pool, feature_map_pool)
